# baseline (speedup 1.0000x reference)
import jax
import jax.numpy as jnp
from jax import lax
from jax.experimental import pallas as pl
from jax.experimental.pallas import tpu as pltpu
from jax.experimental.pallas import tpu_sc as plsc

_NC = 2
_NS = 16
_L = 16
_NW = _NC * _NS

_ROWS, _COLS = 16384, 2048
_SC_ROWS = 4096
_TC_ROWS = _ROWS - _SC_ROWS
_ROWS_PER_W = _SC_ROWS // _NW
_CHUNK_R = 4
_NBUF = 8
_NCHUNKS = _ROWS_PER_W // _CHUNK_R
_CVECS = _COLS // _L
_UNROLL = 4

_mesh = plsc.VectorSubcoreMesh(core_axis_name="c", subcore_axis_name="s")


def _sc_body(x_hbm, vv_hbm, o_hbm, buf, vvv,
             si0, si1, si2, si3, si4, si5, si6, si7,
             so0, so1, so2, so3, so4, so5, so6, so7):
    wid = lax.axis_index("s") * _NC + lax.axis_index("c")
    base = wid * _ROWS_PER_W
    pltpu.sync_copy(vv_hbm, vvv)
    vval = vvv[...]
    sins = (si0, si1, si2, si3, si4, si5, si6, si7)
    souts = (so0, so1, so2, so3, so4, so5, so6, so7)

    def start_in(c, b):
        pltpu.make_async_copy(
            x_hbm.at[pl.ds(base + c * _CHUNK_R, _CHUNK_R)], buf.at[b], sins[b]
        ).start()

    def wait_in(b):
        pltpu.make_async_copy(
            x_hbm.at[pl.ds(base, _CHUNK_R)], buf.at[b], sins[b]
        ).wait()

    def start_out(c, b):
        pltpu.make_async_copy(
            buf.at[b], o_hbm.at[pl.ds(base + c * _CHUNK_R, _CHUNK_R)], souts[b]
        ).start()

    def wait_out(b):
        pltpu.make_async_copy(
            buf.at[b], o_hbm.at[pl.ds(base, _CHUNK_R)], souts[b]
        ).wait()

    for c in range(_NBUF - 1):
        start_in(c, c)

    n_grp = _NCHUNKS // _NBUF

    def outer(gg, _):
        for b in range(_NBUF):
            c = gg * _NBUF + b
            wait_in(b)

            @plsc.parallel_loop(0, _CVECS, 1, unroll=_UNROLL)
            def _(j):
                sl = pl.ds(j * _L, _L)
                for r in range(_CHUNK_R):
                    v = buf[b, r, sl]
                    buf[b, r, sl] = jnp.where(v > 0.5, vval, v)

            start_out(c, b)

            bf = (b + _NBUF - 1) % _NBUF
            if b == 0:
                @pl.when(gg > 0)
                def _():
                    wait_out(bf)
                start_in(c + _NBUF - 1, bf)
            else:
                @pl.when(gg < n_grp - 1)
                def _():
                    wait_out(bf)
                    start_in(c + _NBUF - 1, bf)
        return 0

    lax.fori_loop(0, n_grp, outer, 0, unroll=False)
    for b in range(_NBUF):
        wait_out(b)


_sc_call = pl.kernel(
    _sc_body,
    out_type=jax.ShapeDtypeStruct((_SC_ROWS, _COLS), jnp.float32),
    mesh=_mesh,
    scratch_types=[
        pltpu.VMEM((_NBUF, _CHUNK_R, _COLS), jnp.float32),
        pltpu.VMEM((_L,), jnp.float32),
    ] + [pltpu.SemaphoreType.DMA] * 16,
    compiler_params=pltpu.CompilerParams(use_tc_tiling_on_sc=True),
)

_TC_BLOCK = 1024
_TC_OFF = _SC_ROWS // _TC_BLOCK


def _tc_body(value_ref, x_ref, o_ref):
    x = x_ref[...]
    o_ref[...] = jnp.where(x > 0.5, value_ref[0], x)


def _tc_call(x, v):
    return pl.pallas_call(
        _tc_body,
        grid=(_TC_ROWS // _TC_BLOCK,),
        in_specs=[
            pl.BlockSpec(memory_space=pltpu.SMEM),
            pl.BlockSpec((_TC_BLOCK, _COLS), lambda i: (i + _TC_OFF, 0)),
        ],
        out_specs=pl.BlockSpec((_TC_BLOCK, _COLS), lambda i: (i, 0)),
        out_shape=jax.ShapeDtypeStruct((_TC_ROWS, _COLS), jnp.float32),
    )(v, x)


def kernel(x, value):
    vv = jnp.broadcast_to(jnp.reshape(value, (1,)), (_L,))
    top = _sc_call(x, vv)
    bot = _tc_call(x, jnp.reshape(value, (1,)))
    return jnp.concatenate([top, bot], axis=0)
